# baseline (speedup 1.0000x reference)
import functools
import math

import jax
import jax.numpy as jnp
from jax import lax
from jax.experimental import pallas as pl
from jax.experimental.pallas import tpu as pltpu
from jax.experimental.pallas import tpu_sc as plsc

_EMBEDDING_DIM = 32
_SCALE = math.sqrt(float(_EMBEDDING_DIM))

_NUM_CORES = 2
_NUM_SUBCORES = 16
_NUM_WORKERS = _NUM_CORES * _NUM_SUBCORES
_CHUNK = 128


def _sc_embedding_lookup(tokens_2d, table, n_rows):
    n_chunks_total = tokens_2d.shape[0]
    chunks_per_worker = n_chunks_total // _NUM_WORKERS
    rows_per_worker = chunks_per_worker * _CHUNK
    dim = table.shape[1]

    mesh = plsc.VectorSubcoreMesh(core_axis_name="c", subcore_axis_name="s")

    @functools.partial(
        pl.kernel,
        mesh=mesh,
        out_type=jax.ShapeDtypeStruct((n_rows, dim), jnp.float32),
        scratch_types=[
            pltpu.VMEM((chunks_per_worker, _CHUNK), jnp.int32),
            pltpu.VMEM((_CHUNK, dim), jnp.float32),
            pltpu.SemaphoreType.DMA,
        ],
        compiler_params=pltpu.CompilerParams(use_tc_tiling_on_sc=False),
    )
    def k(tokens_hbm, table_hbm, out_hbm, idx_v, rows_v, sem):
        wid = lax.axis_index("s") * _NUM_CORES + lax.axis_index("c")
        chunk0 = wid * chunks_per_worker
        base_row = wid * rows_per_worker

        pltpu.sync_copy(tokens_hbm.at[pl.ds(chunk0, chunks_per_worker)], idx_v)

        scale = jnp.float32(_SCALE)
        vecs_per_chunk = _CHUNK * dim // 16

        def chunk_body(j, _):
            pltpu.async_copy(table_hbm.at[idx_v.at[j]], rows_v, sem).wait()

            def scale_body(i, _):
                r = i // (dim // 16)
                c = (i % (dim // 16)) * 16
                rows_v[r, pl.ds(c, 16)] = rows_v[r, pl.ds(c, 16)] * scale
                return 0

            lax.fori_loop(0, vecs_per_chunk, scale_body, 0)
            pltpu.sync_copy(
                rows_v, out_hbm.at[pl.ds(base_row + j * _CHUNK, _CHUNK)]
            )
            return 0

        lax.fori_loop(0, chunks_per_worker, chunk_body, 0)

    return k(tokens_2d, table)


def kernel(tokens, table):
    b, s = tokens.shape
    n_rows = b * s
    idx = tokens.reshape(n_rows).astype(jnp.int32)
    tokens_2d = idx.reshape(n_rows // _CHUNK, _CHUNK)
    out = _sc_embedding_lookup(tokens_2d, table, n_rows)
    return out.reshape(b, s, table.shape[1])


# speedup vs baseline: 1.2547x; 1.2547x over previous
import functools
import math

import jax
import jax.numpy as jnp
from jax import lax
from jax.experimental import pallas as pl
from jax.experimental.pallas import tpu as pltpu
from jax.experimental.pallas import tpu_sc as plsc

_EMBEDDING_DIM = 32
_SCALE = math.sqrt(float(_EMBEDDING_DIM))

_NUM_CORES = 2
_NUM_SUBCORES = 16
_NUM_WORKERS = _NUM_CORES * _NUM_SUBCORES
_CHUNK = 128
_NBUF = 4


def _sc_embedding_lookup(tokens_2d, table, n_rows):
    n_chunks_total = tokens_2d.shape[0]
    chunks_per_worker = n_chunks_total // _NUM_WORKERS
    rows_per_worker = chunks_per_worker * _CHUNK
    dim = table.shape[1]
    n_groups = chunks_per_worker // _NBUF

    mesh = plsc.VectorSubcoreMesh(core_axis_name="c", subcore_axis_name="s")

    @functools.partial(
        pl.kernel,
        mesh=mesh,
        out_type=jax.ShapeDtypeStruct((n_rows, dim), jnp.float32),
        scratch_types=[
            pltpu.VMEM((chunks_per_worker, _CHUNK), jnp.int32),
            pltpu.VMEM((_NBUF, _CHUNK, dim), jnp.float32),
            pltpu.VMEM((_NBUF, _CHUNK, dim), jnp.float32),
        ]
        + [pltpu.SemaphoreType.DMA] * (2 * _NBUF),
        compiler_params=pltpu.CompilerParams(use_tc_tiling_on_sc=False),
    )
    def k(tokens_hbm, table_hbm, out_hbm, idx_v, gbuf, sbuf, *sems):
        gsems = sems[:_NBUF]
        ssems = sems[_NBUF:]
        wid = lax.axis_index("s") * _NUM_CORES + lax.axis_index("c")
        chunk0 = wid * chunks_per_worker
        base_row = wid * rows_per_worker

        pltpu.sync_copy(tokens_hbm.at[pl.ds(chunk0, chunks_per_worker)], idx_v)

        scale = jnp.float32(_SCALE)

        def issue_gather(j, b):
            pltpu.async_copy(table_hbm.at[idx_v.at[j]], gbuf.at[b], gsems[b])

        def wait_gather(b):
            pltpu.make_async_copy(
                table_hbm.at[idx_v.at[0]], gbuf.at[b], gsems[b]
            ).wait()

        def issue_store(j, b):
            pltpu.async_copy(
                sbuf.at[b],
                out_hbm.at[pl.ds(base_row + j * _CHUNK, _CHUNK)],
                ssems[b],
            )

        def wait_store(b):
            pltpu.make_async_copy(
                sbuf.at[b], out_hbm.at[pl.ds(base_row, _CHUNK)], ssems[b]
            ).wait()

        for b in range(_NBUF):
            issue_gather(b, b)

        def group_body(g, _):
            for b in range(_NBUF):
                j = g * _NBUF + b
                wait_gather(b)

                @pl.when(g > 0)
                def _():
                    wait_store(b)

                gb = gbuf.at[b]
                sb = sbuf.at[b]

                @plsc.parallel_loop(0, _CHUNK, 1, unroll=4)
                def _(r):
                    for c in range(0, dim, 16):
                        sb[r, pl.ds(c, 16)] = gb[r, pl.ds(c, 16)] * scale

                issue_store(j, b)

                @pl.when(g < n_groups - 1)
                def _():
                    issue_gather(j + _NBUF, b)
            return 0

        lax.fori_loop(0, n_groups, group_body, 0)

        for b in range(_NBUF):
            wait_store(b)

    return k(tokens_2d, table)


def kernel(tokens, table):
    b, s = tokens.shape
    n_rows = b * s
    idx = tokens.reshape(n_rows).astype(jnp.int32)
    tokens_2d = idx.reshape(n_rows // _CHUNK, _CHUNK)
    out = _sc_embedding_lookup(tokens_2d, table, n_rows)
    return out.reshape(b, s, table.shape[1])


# speedup vs baseline: 1.2584x; 1.0029x over previous
import functools
import math

import jax
import jax.numpy as jnp
from jax import lax
from jax.experimental import pallas as pl
from jax.experimental.pallas import tpu as pltpu
from jax.experimental.pallas import tpu_sc as plsc

_EMBEDDING_DIM = 32
_SCALE = math.sqrt(float(_EMBEDDING_DIM))

_NUM_CORES = 2
_NUM_SUBCORES = 16
_NUM_WORKERS = _NUM_CORES * _NUM_SUBCORES
_CHUNK = 128
_NBUF = 8


def _sc_embedding_lookup(tokens_flat, table, n_rows):
    rows_per_worker = n_rows // _NUM_WORKERS
    chunks_per_worker = rows_per_worker // _CHUNK
    dim = table.shape[1]
    n_groups = chunks_per_worker // _NBUF

    mesh = plsc.VectorSubcoreMesh(core_axis_name="c", subcore_axis_name="s")

    @functools.partial(
        pl.kernel,
        mesh=mesh,
        out_type=jax.ShapeDtypeStruct((n_rows, dim), jnp.float32),
        scratch_types=[
            pltpu.VMEM((rows_per_worker,), jnp.int32),
            pltpu.VMEM((_NBUF, _CHUNK, dim), jnp.float32),
            pltpu.VMEM((_NBUF, _CHUNK, dim), jnp.float32),
        ]
        + [pltpu.SemaphoreType.DMA] * (2 * _NBUF),
        compiler_params=pltpu.CompilerParams(use_tc_tiling_on_sc=False),
    )
    def k(tokens_hbm, table_hbm, out_hbm, idx_v, gbuf, sbuf, *sems):
        gsems = sems[:_NBUF]
        ssems = sems[_NBUF:]
        wid = lax.axis_index("s") * _NUM_CORES + lax.axis_index("c")
        base_row = wid * rows_per_worker

        pltpu.sync_copy(tokens_hbm.at[pl.ds(base_row, rows_per_worker)], idx_v)

        scale = jnp.float32(_SCALE)

        def issue_gather(j, b):
            pltpu.async_copy(
                table_hbm.at[idx_v.at[pl.ds(j * _CHUNK, _CHUNK)]],
                gbuf.at[b],
                gsems[b],
            )

        def wait_gather(b):
            pltpu.make_async_copy(
                table_hbm.at[idx_v.at[pl.ds(0, _CHUNK)]], gbuf.at[b], gsems[b]
            ).wait()

        def issue_store(j, b):
            pltpu.async_copy(
                sbuf.at[b],
                out_hbm.at[pl.ds(base_row + j * _CHUNK, _CHUNK)],
                ssems[b],
            )

        def wait_store(b):
            pltpu.make_async_copy(
                sbuf.at[b], out_hbm.at[pl.ds(base_row, _CHUNK)], ssems[b]
            ).wait()

        for b in range(_NBUF):
            issue_gather(b, b)

        def group_body(g, _):
            for b in range(_NBUF):
                j = g * _NBUF + b
                wait_gather(b)

                @pl.when(g > 0)
                def _():
                    wait_store(b)

                gb = gbuf.at[b]
                sb = sbuf.at[b]

                @plsc.parallel_loop(0, _CHUNK, 1, unroll=4)
                def _(r):
                    for c in range(0, dim, 16):
                        sb[r, pl.ds(c, 16)] = gb[r, pl.ds(c, 16)] * scale

                issue_store(j, b)

                @pl.when(g < n_groups - 1)
                def _():
                    issue_gather(j + _NBUF, b)
            return 0

        lax.fori_loop(0, n_groups, group_body, 0)

        for b in range(_NBUF):
            wait_store(b)

    return k(tokens_flat, table)


def kernel(tokens, table):
    b, s = tokens.shape
    n_rows = b * s
    idx = tokens.reshape(n_rows).astype(jnp.int32)
    out = _sc_embedding_lookup(idx, table, n_rows)
    return out.reshape(b, s, table.shape[1])


# speedup vs baseline: 1.8365x; 1.4594x over previous
import functools
import math

import jax
import jax.numpy as jnp
from jax import lax
from jax.experimental import pallas as pl
from jax.experimental.pallas import tpu as pltpu
from jax.experimental.pallas import tpu_sc as plsc

_EMBEDDING_DIM = 32
_SCALE = math.sqrt(float(_EMBEDDING_DIM))

_NUM_CORES = 2
_NUM_SUBCORES = 16
_NUM_WORKERS = _NUM_CORES * _NUM_SUBCORES
_CHUNK = 128
_NBUF = 4
_LANES = 128
_PACK = _LANES // _EMBEDDING_DIM


def _sc_embedding_lookup(tokens_2d, table128, n_rows):
    rows_per_worker = n_rows // _NUM_WORKERS
    chunks_per_worker = rows_per_worker // _CHUNK
    n_groups = chunks_per_worker // _NBUF
    out_rows = n_rows // _PACK
    out_rows_per_worker = rows_per_worker // _PACK
    out_rows_per_chunk = _CHUNK // _PACK

    mesh = plsc.VectorSubcoreMesh(core_axis_name="c", subcore_axis_name="s")

    @functools.partial(
        pl.kernel,
        mesh=mesh,
        out_type=jax.ShapeDtypeStruct((out_rows, _LANES), jnp.float32),
        scratch_types=[
            pltpu.VMEM((chunks_per_worker, _CHUNK), jnp.int32),
            pltpu.VMEM((_NBUF, _CHUNK), jnp.int32),
            pltpu.VMEM((_NBUF, _CHUNK, _LANES), jnp.float32),
            pltpu.VMEM((_NBUF, out_rows_per_chunk, _LANES), jnp.float32),
        ]
        + [pltpu.SemaphoreType.DMA] * (2 * _NBUF),
    )
    def k(tokens_hbm, table_hbm, out_hbm, idx_v, idx4_v, gbuf, sbuf, *sems):
        gsems = sems[:_NBUF]
        ssems = sems[_NBUF:]
        wid = lax.axis_index("s") * _NUM_CORES + lax.axis_index("c")
        base_out = wid * out_rows_per_worker

        pltpu.sync_copy(
            tokens_hbm.at[pl.ds(wid * chunks_per_worker, chunks_per_worker)],
            idx_v,
        )

        scale = jnp.float32(_SCALE)

        def issue_gather(j, b):
            for m in range(_CHUNK // 16):
                idx4_v[b, pl.ds(m * 16, 16)] = (
                    idx_v[j, pl.ds(m * 16, 16)] >> 2
                )
            pltpu.async_copy(
                table_hbm.at[idx4_v.at[b]], gbuf.at[b], gsems[b]
            )

        def wait_gather(b):
            pltpu.make_async_copy(
                table_hbm.at[idx4_v.at[b]], gbuf.at[b], gsems[b]
            ).wait()

        def issue_store(j, b):
            pltpu.async_copy(
                sbuf.at[b],
                out_hbm.at[
                    pl.ds(base_out + j * out_rows_per_chunk, out_rows_per_chunk)
                ],
                ssems[b],
            )

        def wait_store(b):
            pltpu.make_async_copy(
                sbuf.at[b],
                out_hbm.at[pl.ds(base_out, out_rows_per_chunk)],
                ssems[b],
            ).wait()

        for b in range(_NBUF):
            issue_gather(b, b)

        def group_body(g, _):
            for b in range(_NBUF):
                j = g * _NBUF + b
                wait_gather(b)

                @pl.when(g > 0)
                def _():
                    wait_store(b)

                gb = gbuf.at[b]
                sb = sbuf.at[b]
                jv = idx_v.at[j]

                @plsc.parallel_loop(0, _CHUNK // 16, 1)
                def _(m):
                    tvec = jv[pl.ds(m * 16, 16)]
                    cvec = (tvec & 3) * _EMBEDDING_DIM
                    for l in range(16):
                        c = cvec[l]
                        dst_r = m * (16 // _PACK) + l // _PACK
                        dst_c = (l % _PACK) * _EMBEDDING_DIM
                        for h in range(0, _EMBEDDING_DIM, 16):
                            sb[dst_r, pl.ds(dst_c + h, 16)] = (
                                gb[m * 16 + l, pl.ds(c + h, 16)] * scale
                            )

                issue_store(j, b)

                @pl.when(g < n_groups - 1)
                def _():
                    issue_gather(j + _NBUF, b)
            return 0

        lax.fori_loop(0, n_groups, group_body, 0)

        for b in range(_NBUF):
            wait_store(b)

    return k(tokens_2d, table128)


def kernel(tokens, table):
    b, s = tokens.shape
    n_rows = b * s
    dim = table.shape[1]
    idx = tokens.reshape(n_rows // _CHUNK, _CHUNK).astype(jnp.int32)
    table128 = table.reshape(table.shape[0] // _PACK, _LANES)
    out128 = _sc_embedding_lookup(idx, table128, n_rows)
    return out128.reshape(b, s, dim)


# speedup vs baseline: 2.0377x; 1.1096x over previous
import functools
import math

import jax
import jax.numpy as jnp
from jax import lax
from jax.experimental import pallas as pl
from jax.experimental.pallas import tpu as pltpu
from jax.experimental.pallas import tpu_sc as plsc

_EMBEDDING_DIM = 32
_SCALE = math.sqrt(float(_EMBEDDING_DIM))

_NUM_CORES = 2
_NUM_SUBCORES = 16
_NUM_WORKERS = _NUM_CORES * _NUM_SUBCORES
_CHUNK = 128
_NBUF = 4
_LANES = 128
_PACK = _LANES // _EMBEDDING_DIM


def _sc_embedding_lookup(tokens_2d, table, n_rows):
    rows_per_worker = n_rows // _NUM_WORKERS
    chunks_per_worker = rows_per_worker // _CHUNK
    n_groups = chunks_per_worker // _NBUF
    out_rows = n_rows // _PACK
    out_rows_per_worker = rows_per_worker // _PACK
    out_rows_per_chunk = _CHUNK // _PACK
    dim = table.shape[1]

    mesh = plsc.VectorSubcoreMesh(core_axis_name="c", subcore_axis_name="s")

    @functools.partial(
        pl.kernel,
        mesh=mesh,
        out_type=jax.ShapeDtypeStruct((out_rows, _LANES), jnp.float32),
        scratch_types=[
            pltpu.VMEM((chunks_per_worker, _CHUNK), jnp.int32),
            pltpu.VMEM((_NBUF, _CHUNK, dim), jnp.float32),
            pltpu.VMEM((_NBUF, out_rows_per_chunk, _LANES), jnp.float32),
        ]
        + [pltpu.SemaphoreType.DMA] * (2 * _NBUF),
        compiler_params=pltpu.CompilerParams(use_tc_tiling_on_sc=False),
    )
    def k(tokens_hbm, table_hbm, out_hbm, idx_v, gbuf, sbuf, *sems):
        gsems = sems[:_NBUF]
        ssems = sems[_NBUF:]
        wid = lax.axis_index("s") * _NUM_CORES + lax.axis_index("c")
        base_out = wid * out_rows_per_worker

        pltpu.sync_copy(
            tokens_hbm.at[pl.ds(wid * chunks_per_worker, chunks_per_worker)],
            idx_v,
        )

        scale = jnp.float32(_SCALE)

        def issue_gather(j, b):
            pltpu.async_copy(table_hbm.at[idx_v.at[j]], gbuf.at[b], gsems[b])

        def wait_gather(b):
            pltpu.make_async_copy(
                table_hbm.at[idx_v.at[0]], gbuf.at[b], gsems[b]
            ).wait()

        def issue_store(j, b):
            pltpu.async_copy(
                sbuf.at[b],
                out_hbm.at[
                    pl.ds(base_out + j * out_rows_per_chunk, out_rows_per_chunk)
                ],
                ssems[b],
            )

        def wait_store(b):
            pltpu.make_async_copy(
                sbuf.at[b],
                out_hbm.at[pl.ds(base_out, out_rows_per_chunk)],
                ssems[b],
            ).wait()

        for b in range(_NBUF):
            issue_gather(b, b)

        def group_body(g, _):
            for b in range(_NBUF):
                j = g * _NBUF + b
                wait_gather(b)

                @pl.when(g > 0)
                def _():
                    wait_store(b)

                gb = gbuf.at[b]
                sb = sbuf.at[b]

                @plsc.parallel_loop(0, _CHUNK, 1, unroll=4)
                def _(t):
                    dst_r = t // _PACK
                    dst_c = (t % _PACK) * _EMBEDDING_DIM
                    for h in range(0, _EMBEDDING_DIM, 16):
                        sb[dst_r, pl.ds(dst_c + h, 16)] = (
                            gb[t, pl.ds(h, 16)] * scale
                        )

                issue_store(j, b)

                @pl.when(g < n_groups - 1)
                def _():
                    issue_gather(j + _NBUF, b)
            return 0

        lax.fori_loop(0, n_groups, group_body, 0)

        for b in range(_NBUF):
            wait_store(b)

    return k(tokens_2d, table)


def kernel(tokens, table):
    b, s = tokens.shape
    n_rows = b * s
    dim = table.shape[1]
    idx = tokens.reshape(n_rows // _CHUNK, _CHUNK).astype(jnp.int32)
    out128 = _sc_embedding_lookup(idx, table, n_rows)
    return out128.reshape(b, s, dim)
